# baseline (speedup 1.0000x reference)
import functools

import jax
import jax.numpy as jnp
from jax import lax
from jax.experimental import pallas as pl
from jax.experimental.pallas import tpu as pltpu
from jax.experimental.pallas import tpu_sc as plsc

Q = 1024
D = 128
KB = 2048
G = 128
GPB = KB // G
NEG = -1e30


def _tc_body(nkb, k_real, q_ref, k_ref, v_ref, kt_ref, vt_ref,
             wv_ref, sims_ref, gmax_ref, acc_ref, l_ref, qn_ref):
    kstep = pl.program_id(0)

    @pl.when(kstep == 0)
    def _init():
        qn = jnp.sum(jnp.abs(q_ref[...]), axis=1, keepdims=True)
        qn_ref[...] = 1.0 / jnp.sqrt(qn)
        acc_ref[...] = jnp.zeros_like(acc_ref)
        l_ref[...] = jnp.zeros_like(l_ref)

    def _scores(kb):
        s_raw = jax.lax.dot_general(q_ref[...], kb, (((1,), (1,)), ((), ())),
                                    preferred_element_type=jnp.float32)
        kn = jax.lax.dot_general(jnp.ones((1, D), jnp.float32), jnp.abs(kb),
                                 (((1,), (1,)), ((), ())),
                                 precision=jax.lax.Precision.HIGHEST,
                                 preferred_element_type=jnp.float32)
        kn = jnp.maximum(kn, 1e-30)
        return (s_raw * (1.0 / jnp.sqrt(kn))) * qn_ref[...]

    def _tail(sv, vb):
        for j in range(GPB):
            sl = sv[:, j * G:(j + 1) * G]
            sims_ref[j * Q:(j + 1) * Q, :] = sl
            gmax_ref[0, :, j:j + 1] = jnp.max(sl, axis=1, keepdims=True)
        p = jnp.exp(sv)
        l_ref[...] += jax.lax.dot_general(
            p, jnp.ones((KB, 1), jnp.float32), (((1,), (0,)), ((), ())),
            preferred_element_type=jnp.float32)
        acc_ref[...] += jax.lax.dot_general(
            p, vb, (((1,), (0,)), ((), ())),
            preferred_element_type=jnp.float32)

    @pl.when(kstep == nkb - 1)
    def _tail_masked():
        s = _scores(kt_ref[...])
        col = kstep * KB + jax.lax.broadcasted_iota(jnp.int32, (Q, KB), 1)
        _tail(jnp.where(col < k_real, s, NEG), vt_ref[...])

    @pl.when(kstep < nkb - 1)
    def _tail_plain():
        _tail(_scores(k_ref[...]), v_ref[...])

    @pl.when(kstep == nkb - 1)
    def _finish():
        wv_ref[...] = acc_ref[...] / l_ref[...]


def _tc_flash(queries, keys, values):
    k_real = keys.shape[0]
    nkb = (k_real + KB - 1) // KB
    kp = nkb * KB
    tail0 = (nkb - 1) * KB
    ktail = jnp.zeros((KB, D), jnp.float32).at[:k_real - tail0].set(keys[tail0:])
    vtail = jnp.zeros((KB, D), jnp.float32).at[:k_real - tail0].set(values[tail0:])
    nmain = nkb - 1
    wv, sims, gmax = pl.pallas_call(
        functools.partial(_tc_body, nkb, k_real),
        grid=(nkb,),
        in_specs=[
            pl.BlockSpec((Q, D), lambda k: (0, 0)),
            pl.BlockSpec((KB, D), lambda k: (jnp.minimum(k, nmain - 1), 0)),
            pl.BlockSpec((KB, D), lambda k: (jnp.minimum(k, nmain - 1), 0)),
            pl.BlockSpec((KB, D), lambda k: (0, 0)),
            pl.BlockSpec((KB, D), lambda k: (0, 0)),
        ],
        out_specs=[
            pl.BlockSpec((Q, D), lambda k: (0, 0)),
            pl.BlockSpec((GPB * Q, G), lambda k: (k, 0)),
            pl.BlockSpec((1, Q, GPB), lambda k: (k, 0, 0)),
        ],
        out_shape=[
            jax.ShapeDtypeStruct((Q, D), jnp.float32),
            jax.ShapeDtypeStruct((kp // G * Q, G), jnp.float32),
            jax.ShapeDtypeStruct((nkb, Q, GPB), jnp.float32),
        ],
        scratch_shapes=[
            pltpu.VMEM((Q, D), jnp.float32),
            pltpu.VMEM((Q, 1), jnp.float32),
            pltpu.VMEM((Q, 1), jnp.float32),
        ],
        compiler_params=pltpu.CompilerParams(
            dimension_semantics=("arbitrary",)),
    )(queries, keys, values, ktail, vtail)
    return wv, sims, gmax


NC, NS, L = 2, 16, 16
NW = NC * NS
QPW = Q // NW
TOP = 16


def _merge16(Rv, Ri, S, Si):
    Ss = plsc.sort_key_val(S, Si, descending=False)
    Sv, Svi = Ss
    take_r = Rv >= Sv
    Lv = jnp.where(take_r, Rv, Sv)
    Li = jnp.where(take_r, Ri, Svi)
    Ls = plsc.sort_key_val(Lv, Li, descending=True)
    return Ls[0], Ls[1]


def _sc_topk(gmax2, sims_rows):
    nkb = gmax2.shape[0]
    nsub = GPB // L
    nch = nkb * nsub

    mesh = plsc.VectorSubcoreMesh(core_axis_name="c", subcore_axis_name="s")

    @functools.partial(
        pl.kernel,
        out_type=[
            jax.ShapeDtypeStruct((Q, TOP), jnp.float32),
            jax.ShapeDtypeStruct((Q, TOP), jnp.int32),
        ],
        mesh=mesh,
        compiler_params=pltpu.CompilerParams(needs_layout_passes=False),
        scratch_types=[
            pltpu.VMEM((nkb, QPW * GPB), jnp.float32),
            pltpu.VMEM((QPW * TOP,), jnp.int32),
            pltpu.VMEM((QPW * TOP, G), jnp.float32),
            pltpu.VMEM((QPW, TOP), jnp.float32),
            pltpu.VMEM((QPW, TOP), jnp.int32),
            pltpu.SemaphoreType.DMA,
        ],
    )
    def sc_kernel(gmax_hbm, rows_hbm, ts_hbm, ti_hbm,
                  gmax_v, idx_v, rows_v, ts_v, ti_v, sem):
        wid = lax.axis_index("s") * NC + lax.axis_index("c")
        q0 = wid * QPW
        pltpu.sync_copy(gmax_hbm.at[:, pl.ds(q0 * GPB, QPW * GPB)], gmax_v)

        def phase1(qi, _):
            def chunk(t, carry):
                Rv, Ri = carry
                c = t // nsub
                h = t % nsub
                S = gmax_v[c, pl.ds(qi * GPB + h * L, L)]

                def do(cc):
                    Si = t * L + lax.iota(jnp.int32, L)
                    return _merge16(cc[0], cc[1], S, Si)

                return lax.cond(jnp.max(S) > Rv[L - 1], do, lambda cc: cc,
                                (Rv, Ri))

            Rv = jnp.full((L,), NEG, jnp.float32)
            Ri = jnp.zeros((L,), jnp.int32)
            Rv, Ri = lax.fori_loop(0, nch, chunk, (Rv, Ri))
            ts_v[qi, :] = Rv
            ti_v[qi, :] = Ri
            idx_v[pl.ds(qi * TOP, TOP)] = Ri * Q + (q0 + qi)
            return 0

        lax.fori_loop(0, QPW, phase1, 0)

        nrow = QPW * TOP
        for g in range(0, nrow, 128):
            pltpu.async_copy(rows_hbm.at[idx_v.at[pl.ds(g, 128)]],
                             rows_v.at[pl.ds(g, 128)], sem).wait()

        def phase2(qi, _):
            rmax_row = ts_v[qi, :]
            bid_row = ti_v[qi, :]
            carry = (jnp.full((L,), NEG, jnp.float32),
                     jnp.zeros((L,), jnp.int32))
            for r in range(TOP):
                rmax = rmax_row[r]
                bid = bid_row[r]

                def process(carry2, _r=r, _bid=bid):
                    def chunk(cj, carry3):
                        Rv, Ri = carry3
                        S = rows_v[qi * TOP + _r, pl.ds(cj * L, L)]
                        Si = _bid * G + cj * L + lax.iota(jnp.int32, L)
                        return _merge16(Rv, Ri, S, Si)

                    return lax.fori_loop(0, G // L, chunk, carry2)

                t10 = carry[0][9]
                carry = lax.cond(rmax > t10, process, lambda c: c, carry)
            ts_v[qi, :] = carry[0]
            ti_v[qi, :] = carry[1]
            return 0

        lax.fori_loop(0, QPW, phase2, 0)
        pltpu.sync_copy(ts_v, ts_hbm.at[pl.ds(q0, QPW)])
        pltpu.sync_copy(ti_v, ti_hbm.at[pl.ds(q0, QPW)])

    return sc_kernel(gmax2, sims_rows)


def kernel(queries, keys, values):
    wv, sims_rows, gmax3 = _tc_flash(queries, keys, values)
    gmax2 = jnp.reshape(gmax3, (gmax3.shape[0], Q * GPB))
    ts, ti = _sc_topk(gmax2, sims_rows)
    return wv, ts[:, :10], ti[:, :10]


# speedup vs baseline: 1.0925x; 1.0925x over previous
import functools

import jax
import jax.numpy as jnp
from jax import lax
from jax.experimental import pallas as pl
from jax.experimental.pallas import tpu as pltpu
from jax.experimental.pallas import tpu_sc as plsc

Q = 1024
D = 128
KB = 2048
G = 128
GPB = KB // G
NEG = -1e30


def _tc_body(nkb, k_real, q_ref, k_ref, v_ref, kt_ref, vt_ref,
             wv_ref, sims_ref, gmax_ref, acc_ref, l_ref, qn_ref):
    kstep = pl.program_id(0)

    @pl.when(kstep == 0)
    def _init():
        qn = jnp.sum(jnp.abs(q_ref[...]), axis=1, keepdims=True)
        qn_ref[...] = 1.0 / jnp.sqrt(qn)
        acc_ref[...] = jnp.zeros_like(acc_ref)
        l_ref[...] = jnp.zeros_like(l_ref)

    def _scores(kb):
        s_raw = jax.lax.dot_general(q_ref[...], kb, (((1,), (1,)), ((), ())),
                                    preferred_element_type=jnp.float32)
        kn = jax.lax.dot_general(jnp.ones((1, D), jnp.float32), jnp.abs(kb),
                                 (((1,), (1,)), ((), ())),
                                 precision=jax.lax.Precision.HIGHEST,
                                 preferred_element_type=jnp.float32)
        kn = jnp.maximum(kn, 1e-30)
        return (s_raw * (1.0 / jnp.sqrt(kn))) * qn_ref[...]

    def _tail(sv, vb):
        for j in range(GPB):
            sl = sv[:, j * G:(j + 1) * G]
            sims_ref[j * Q:(j + 1) * Q, :] = sl
            gmax_ref[0, :, j:j + 1] = jnp.max(sl, axis=1, keepdims=True)
        p = jnp.exp(sv)
        l_ref[...] += jax.lax.dot_general(
            p, jnp.ones((KB, 1), jnp.float32), (((1,), (0,)), ((), ())),
            preferred_element_type=jnp.float32)
        acc_ref[...] += jax.lax.dot_general(
            p, vb, (((1,), (0,)), ((), ())),
            preferred_element_type=jnp.float32)

    @pl.when(kstep == nkb - 1)
    def _tail_masked():
        s = _scores(kt_ref[...])
        col = kstep * KB + jax.lax.broadcasted_iota(jnp.int32, (Q, KB), 1)
        _tail(jnp.where(col < k_real, s, NEG), vt_ref[...])

    @pl.when(kstep < nkb - 1)
    def _tail_plain():
        _tail(_scores(k_ref[...]), v_ref[...])

    @pl.when(kstep == nkb - 1)
    def _finish():
        wv_ref[...] = acc_ref[...] / l_ref[...]


def _tc_flash(queries, keys, values):
    k_real = keys.shape[0]
    nkb = (k_real + KB - 1) // KB
    kp = nkb * KB
    tail0 = (nkb - 1) * KB
    ktail = jnp.zeros((KB, D), jnp.float32).at[:k_real - tail0].set(keys[tail0:])
    vtail = jnp.zeros((KB, D), jnp.float32).at[:k_real - tail0].set(values[tail0:])
    nmain = nkb - 1
    wv, sims, gmax = pl.pallas_call(
        functools.partial(_tc_body, nkb, k_real),
        grid=(nkb,),
        in_specs=[
            pl.BlockSpec((Q, D), lambda k: (0, 0)),
            pl.BlockSpec((KB, D), lambda k: (jnp.minimum(k, nmain - 1), 0)),
            pl.BlockSpec((KB, D), lambda k: (jnp.minimum(k, nmain - 1), 0)),
            pl.BlockSpec((KB, D), lambda k: (0, 0)),
            pl.BlockSpec((KB, D), lambda k: (0, 0)),
        ],
        out_specs=[
            pl.BlockSpec((Q, D), lambda k: (0, 0)),
            pl.BlockSpec((GPB * Q, G), lambda k: (k, 0)),
            pl.BlockSpec((1, Q, GPB), lambda k: (k, 0, 0)),
        ],
        out_shape=[
            jax.ShapeDtypeStruct((Q, D), jnp.float32),
            jax.ShapeDtypeStruct((kp // G * Q, G), jnp.float32),
            jax.ShapeDtypeStruct((nkb, Q, GPB), jnp.float32),
        ],
        scratch_shapes=[
            pltpu.VMEM((Q, D), jnp.float32),
            pltpu.VMEM((Q, 1), jnp.float32),
            pltpu.VMEM((Q, 1), jnp.float32),
        ],
        compiler_params=pltpu.CompilerParams(
            dimension_semantics=("arbitrary",)),
    )(queries, keys, values, ktail, vtail)
    return wv, sims, gmax


NC, NS, L = 2, 16, 16
NW = NC * NS
QPW = Q // NW
TOP = 16


def _merge16(Rv, Ri, S, Si):
    Ss = plsc.sort_key_val(S, Si, descending=False)
    Sv, Svi = Ss
    take_r = Rv >= Sv
    Lv = jnp.where(take_r, Rv, Sv)
    Li = jnp.where(take_r, Ri, Svi)
    Ls = plsc.sort_key_val(Lv, Li, descending=True)
    return Ls[0], Ls[1]


def _sc_topk(gmax2, sims_rows):
    nkb = gmax2.shape[0]
    nsub = GPB // L
    nch = nkb * nsub

    mesh = plsc.VectorSubcoreMesh(core_axis_name="c", subcore_axis_name="s")

    @functools.partial(
        pl.kernel,
        out_type=[
            jax.ShapeDtypeStruct((Q, TOP), jnp.float32),
            jax.ShapeDtypeStruct((Q, TOP), jnp.int32),
        ],
        mesh=mesh,
        compiler_params=pltpu.CompilerParams(needs_layout_passes=False),
        scratch_types=[
            pltpu.VMEM((nkb, QPW * GPB), jnp.float32),
            pltpu.VMEM((QPW * TOP,), jnp.int32),
            pltpu.VMEM((QPW * TOP, G), jnp.float32),
            pltpu.VMEM((QPW, TOP), jnp.float32),
            pltpu.VMEM((QPW, TOP), jnp.int32),
            pltpu.SemaphoreType.DMA,
        ],
    )
    def sc_kernel(gmax_hbm, rows_hbm, ts_hbm, ti_hbm,
                  gmax_v, idx_v, rows_v, ts_v, ti_v, sem):
        wid = lax.axis_index("s") * NC + lax.axis_index("c")
        q0 = wid * QPW
        pltpu.sync_copy(gmax_hbm.at[:, pl.ds(q0 * GPB, QPW * GPB)], gmax_v)

        def phase1(qi, _):
            def load(t):
                c = t // nsub
                h = t % nsub
                S = gmax_v[c, pl.ds(qi * GPB + h * L, L)]
                return S, t * L + lax.iota(jnp.int32, L)

            def pair(u, carry):
                Rv0, Ri0, Rv1, Ri1 = carry
                S0, Si0 = load(2 * u)
                S1, Si1 = load(2 * u + 1)
                Rv0, Ri0 = _merge16(Rv0, Ri0, S0, Si0)
                Rv1, Ri1 = _merge16(Rv1, Ri1, S1, Si1)
                return Rv0, Ri0, Rv1, Ri1

            Rv = jnp.full((L,), NEG, jnp.float32)
            Ri = jnp.zeros((L,), jnp.int32)
            Rv0, Ri0, Rv1, Ri1 = lax.fori_loop(0, nch // 2, pair,
                                               (Rv, Ri, Rv, Ri))
            for t in range(nch // 2 * 2, nch):
                S, Si = load(t)
                Rv0, Ri0 = _merge16(Rv0, Ri0, S, Si)
            Rv1r = lax.rev(Rv1, (0,))
            Ri1r = lax.rev(Ri1, (0,))
            take0 = Rv0 >= Rv1r
            Lv = jnp.where(take0, Rv0, Rv1r)
            Li = jnp.where(take0, Ri0, Ri1r)
            Rv, Ri = plsc.sort_key_val(Lv, Li, descending=True)
            ts_v[qi, :] = Rv
            ti_v[qi, :] = Ri
            idx_v[pl.ds(qi * TOP, TOP)] = Ri * Q + (q0 + qi)
            return 0

        lax.fori_loop(0, QPW, phase1, 0)

        nrow = QPW * TOP
        for g in range(0, nrow, 128):
            pltpu.async_copy(rows_hbm.at[idx_v.at[pl.ds(g, 128)]],
                             rows_v.at[pl.ds(g, 128)], sem).wait()

        def phase2(qi, _):
            rmax_row = ts_v[qi, :]
            bid_row = ti_v[qi, :]
            carry = (jnp.full((L,), NEG, jnp.float32),
                     jnp.zeros((L,), jnp.int32))
            for r in range(TOP):
                rmax = rmax_row[r]
                bid = bid_row[r]

                def process(carry2, _r=r, _bid=bid):
                    def chunk(cj, carry3):
                        Rv, Ri = carry3
                        S = rows_v[qi * TOP + _r, pl.ds(cj * L, L)]
                        Si = _bid * G + cj * L + lax.iota(jnp.int32, L)
                        return _merge16(Rv, Ri, S, Si)

                    return lax.fori_loop(0, G // L, chunk, carry2)

                t10 = carry[0][9]
                carry = lax.cond(rmax > t10, process, lambda c: c, carry)
            ts_v[qi, :] = carry[0]
            ti_v[qi, :] = carry[1]
            return 0

        lax.fori_loop(0, QPW, phase2, 0)
        pltpu.sync_copy(ts_v, ts_hbm.at[pl.ds(q0, QPW)])
        pltpu.sync_copy(ti_v, ti_hbm.at[pl.ds(q0, QPW)])

    return sc_kernel(gmax2, sims_rows)


def kernel(queries, keys, values):
    wv, sims_rows, gmax3 = _tc_flash(queries, keys, values)
    gmax2 = jnp.reshape(gmax3, (gmax3.shape[0], Q * GPB))
    ts, ti = _sc_topk(gmax2, sims_rows)
    return wv, ts[:, :10], ti[:, :10]
